# baseline (speedup 1.0000x reference)
import functools

import jax
import jax.numpy as jnp
from jax import lax
from jax.experimental import pallas as pl
from jax.experimental.pallas import tpu as pltpu
from jax.experimental.pallas import tpu_sc as plsc

_NUM_CLS = 1000
_TOPZ = 4096
_N = 16384
_D = 4096

_CR = _N // 128
_RANK_ROWS = 8

_NC = 2
_NS = 16
_NW = _NC * _NS
_SC_LANES = 16

_PAD_OUT = _TOPZ + 8


def _rank_body(ci_ref, cf_ref, rank_ref):
    ib = pl.program_id(0)
    ci3 = ci_ref[...][:, :, None]
    acc0 = jnp.zeros((_RANK_ROWS, 128), jnp.int32)

    def body_before(jr, acc):
        jv3 = cf_ref[pl.ds(jr, 1), :].reshape(1, 1, 128)
        return acc + jnp.sum((jv3 >= ci3).astype(jnp.int32), axis=2)

    def body_after(jr, acc):
        jv3 = cf_ref[pl.ds(jr, 1), :].reshape(1, 1, 128)
        return acc + jnp.sum((jv3 > ci3).astype(jnp.int32), axis=2)

    acc = lax.fori_loop(0, _RANK_ROWS * ib, body_before, acc0)
    acc = lax.fori_loop(_RANK_ROWS * (ib + 1), _CR, body_after, acc)

    r_iota = lax.broadcasted_iota(jnp.int32, (_RANK_ROWS, 128, 1), 0)
    c_iota = lax.broadcasted_iota(jnp.int32, (_RANK_ROWS, 128, 1), 1)
    iidx3 = 128 * (_RANK_ROWS * ib + r_iota) + c_iota
    jlane = lax.broadcasted_iota(jnp.int32, (1, 1, 128), 2)
    for dr in range(_RANK_ROWS):
        jr = _RANK_ROWS * ib + dr
        jv3 = cf_ref[pl.ds(jr, 1), :].reshape(1, 1, 128)
        jidx3 = 128 * jr + jlane
        above = (jv3 > ci3) | ((jv3 == ci3) & (jidx3 < iidx3))
        acc = acc + jnp.sum(above.astype(jnp.int32), axis=2)
    rank_ref[...] = acc


def _rank_call(conf2d):
    return pl.pallas_call(
        _rank_body,
        grid=(_CR // _RANK_ROWS,),
        in_specs=[
            pl.BlockSpec((_RANK_ROWS, 128), lambda i: (i, 0)),
            pl.BlockSpec((_CR, 128), lambda i: (0, 0)),
        ],
        out_specs=pl.BlockSpec((_RANK_ROWS, 128), lambda i: (i, 0)),
        out_shape=jax.ShapeDtypeStruct((_CR, 128), jnp.int32),
    )(conf2d, conf2d)


def _sc_scatter_body(rank_hbm, pred_hbm, yt_hbm, sel_hbm, predg_hbm, ytg_hbm,
                     rank_v, pred_v, yt_v, idx_v, val_v, sem):
    wid = lax.axis_index("s") * _NC + lax.axis_index("c")
    per = _N // _NW
    base = wid * per
    pltpu.sync_copy(rank_hbm.at[pl.ds(base, per)], rank_v)
    pltpu.sync_copy(pred_hbm.at[pl.ds(base, per)], pred_v)
    pltpu.sync_copy(yt_hbm.at[pl.ds(base, per)], yt_v)
    lane = lax.iota(jnp.int32, _SC_LANES)
    for chunk in range(per // 128):
        for k in range(128 // _SC_LANES):
            off = 128 * chunk + _SC_LANES * k
            r = rank_v[pl.ds(off, _SC_LANES)]
            idx_v[pl.ds(_SC_LANES * k, _SC_LANES)] = jnp.where(
                r < _TOPZ, r, _TOPZ)
            val_v[pl.ds(_SC_LANES * k, _SC_LANES)] = base + off + lane
        pltpu.async_copy(val_v, sel_hbm.at[idx_v], sem).wait()
        for k in range(128 // _SC_LANES):
            off = 128 * chunk + _SC_LANES * k
            val_v[pl.ds(_SC_LANES * k, _SC_LANES)] = pred_v[
                pl.ds(off, _SC_LANES)]
        pltpu.async_copy(val_v, predg_hbm.at[idx_v], sem).wait()
        for k in range(128 // _SC_LANES):
            off = 128 * chunk + _SC_LANES * k
            val_v[pl.ds(_SC_LANES * k, _SC_LANES)] = yt_v[
                pl.ds(off, _SC_LANES)]
        pltpu.async_copy(val_v, ytg_hbm.at[idx_v], sem).wait()


def _sc_scatter(rank, pred, y_true):
    mesh = plsc.VectorSubcoreMesh(core_axis_name="c", subcore_axis_name="s")
    per = _N // _NW
    f = pl.kernel(
        _sc_scatter_body,
        out_type=[jax.ShapeDtypeStruct((_PAD_OUT,), jnp.int32)] * 3,
        mesh=mesh,
        scratch_types=[
            pltpu.VMEM((per,), jnp.int32),
            pltpu.VMEM((per,), jnp.int32),
            pltpu.VMEM((per,), jnp.int32),
            pltpu.VMEM((128,), jnp.int32),
            pltpu.VMEM((128,), jnp.int32),
            pltpu.SemaphoreType.DMA,
        ],
    )
    return f(rank, pred, y_true)


def _sc_gather_body(x_hbm, sel_hbm, xg_hbm, idx_v, buf_v, sem):
    wid = lax.axis_index("s") * _NC + lax.axis_index("c")
    rows = _TOPZ // _NW
    base = wid * rows
    pltpu.sync_copy(sel_hbm.at[pl.ds(base, rows)], idx_v)
    chunk = 8
    for c in range(rows // chunk):
        pltpu.async_copy(
            x_hbm.at[idx_v.at[pl.ds(c * chunk, chunk)]], buf_v, sem).wait()
        pltpu.sync_copy(buf_v, xg_hbm.at[pl.ds(base + c * chunk, chunk)])


def _sc_gather(x, sel_pad):
    mesh = plsc.VectorSubcoreMesh(core_axis_name="c", subcore_axis_name="s")
    f = pl.kernel(
        _sc_gather_body,
        out_type=jax.ShapeDtypeStruct((_TOPZ, _D), jnp.float32),
        mesh=mesh,
        scratch_types=[
            pltpu.VMEM((_TOPZ // _NW,), jnp.int32),
            pltpu.VMEM((8, _D), jnp.float32),
            pltpu.SemaphoreType.DMA,
        ],
    )
    return f(x, sel_pad)


_BM = 256


def _loss_body(x_ref, w_ref, pred_ref, yt_ref, loss_ref, acc_ref):
    i = pl.program_id(0)
    logits = jnp.dot(x_ref[...], w_ref[...],
                     preferred_element_type=jnp.float32)
    m = jnp.max(logits, axis=1, keepdims=True)
    e = jnp.exp(logits - m)
    s = jnp.sum(e, axis=1, keepdims=True)
    lse = jnp.log(s) + m
    predg = pred_ref[...]
    cls = lax.broadcasted_iota(jnp.int32, (_BM, _NUM_CLS), 1)
    pick = jnp.sum(jnp.where(cls == predg[:, None], logits, 0.0), axis=1)
    part_loss = jnp.sum(lse[:, 0] - pick)
    part_acc = jnp.sum((predg == yt_ref[...]).astype(jnp.float32))

    @pl.when(i == 0)
    def _():
        loss_ref[...] = jnp.zeros((1, 1), jnp.float32)
        acc_ref[...] = jnp.zeros((1, 1), jnp.float32)

    loss_ref[...] = loss_ref[...] + part_loss
    acc_ref[...] = acc_ref[...] + part_acc

    @pl.when(i == _TOPZ // _BM - 1)
    def _():
        total = loss_ref[...]
        loss_ref[...] = jnp.where(total == 0.0, 0.0,
                                  total / jnp.float32(_TOPZ))
        acc_ref[...] = acc_ref[...] / jnp.float32(_TOPZ)


def _loss_call(xg, W, predg, ytg):
    return pl.pallas_call(
        _loss_body,
        grid=(_TOPZ // _BM,),
        in_specs=[
            pl.BlockSpec((_BM, _D), lambda i: (i, 0)),
            pl.BlockSpec((_D, _NUM_CLS), lambda i: (0, 0)),
            pl.BlockSpec((_BM,), lambda i: (i,)),
            pl.BlockSpec((_BM,), lambda i: (i,)),
        ],
        out_specs=[
            pl.BlockSpec((1, 1), lambda i: (0, 0)),
            pl.BlockSpec((1, 1), lambda i: (0, 0)),
        ],
        out_shape=[
            jax.ShapeDtypeStruct((1, 1), jnp.float32),
            jax.ShapeDtypeStruct((1, 1), jnp.float32),
        ],
    )(xg, W, predg, ytg)


def kernel(x, y_output, y_true, W):
    y_probs = jax.nn.softmax(y_output, axis=1)
    pred = jnp.argmax(y_probs, axis=1).astype(jnp.int32)
    conf = jnp.max(y_probs, axis=1)

    rank2d = _rank_call(conf.reshape(_CR, 128))
    rank = rank2d.reshape(_N)

    sel_pad, predg_pad, ytg_pad = _sc_scatter(rank, pred, y_true)
    selected_idx = sel_pad[:_TOPZ]

    xg = _sc_gather(x, sel_pad)

    loss2, accu2 = _loss_call(xg, W, predg_pad[:_TOPZ], ytg_pad[:_TOPZ])
    return (loss2[0, 0], accu2[0, 0], selected_idx)


# speedup vs baseline: 1.0036x; 1.0036x over previous
import functools

import jax
import jax.numpy as jnp
from jax import lax
from jax.experimental import pallas as pl
from jax.experimental.pallas import tpu as pltpu
from jax.experimental.pallas import tpu_sc as plsc

_NUM_CLS = 1000
_TOPZ = 4096
_N = 16384
_D = 4096

_CR = _N // 128
_RANK_ROWS = 8

_NC = 2
_NS = 16
_NW = _NC * _NS
_SC_LANES = 16

_PAD_OUT = _TOPZ + 8


def _rank_body(ci_ref, cf_ref, rank_ref):
    ib = pl.program_id(0)
    ci3 = ci_ref[...][:, :, None]
    acc0 = jnp.zeros((_RANK_ROWS, 128), jnp.int32)

    def body_before(jr, acc):
        jv3 = cf_ref[pl.ds(jr, 1), :].reshape(1, 1, 128)
        return acc + jnp.sum((jv3 >= ci3).astype(jnp.int32), axis=2)

    def body_after(jr, acc):
        jv3 = cf_ref[pl.ds(jr, 1), :].reshape(1, 1, 128)
        return acc + jnp.sum((jv3 > ci3).astype(jnp.int32), axis=2)

    acc = lax.fori_loop(0, _RANK_ROWS * ib, body_before, acc0)
    acc = lax.fori_loop(_RANK_ROWS * (ib + 1), _CR, body_after, acc)

    r_iota = lax.broadcasted_iota(jnp.int32, (_RANK_ROWS, 128, 1), 0)
    c_iota = lax.broadcasted_iota(jnp.int32, (_RANK_ROWS, 128, 1), 1)
    iidx3 = 128 * (_RANK_ROWS * ib + r_iota) + c_iota
    jlane = lax.broadcasted_iota(jnp.int32, (1, 1, 128), 2)
    for dr in range(_RANK_ROWS):
        jr = _RANK_ROWS * ib + dr
        jv3 = cf_ref[pl.ds(jr, 1), :].reshape(1, 1, 128)
        jidx3 = 128 * jr + jlane
        above = (jv3 > ci3) | ((jv3 == ci3) & (jidx3 < iidx3))
        acc = acc + jnp.sum(above.astype(jnp.int32), axis=2)
    rank_ref[...] = acc


def _rank_call(conf2d):
    return pl.pallas_call(
        _rank_body,
        grid=(_CR // _RANK_ROWS,),
        in_specs=[
            pl.BlockSpec((_RANK_ROWS, 128), lambda i: (i, 0)),
            pl.BlockSpec((_CR, 128), lambda i: (0, 0)),
        ],
        out_specs=pl.BlockSpec((_RANK_ROWS, 128), lambda i: (i, 0)),
        out_shape=jax.ShapeDtypeStruct((_CR, 128), jnp.int32),
    )(conf2d, conf2d)


def _sc_scatter_body(rank_hbm, pred_hbm, yt_hbm, sel_hbm, predg_hbm, ytg_hbm,
                     rank_v, pred_v, yt_v, val_v,
                     idx0, idx1, idx2, idx3, sem):
    wid = lax.axis_index("s") * _NC + lax.axis_index("c")
    per = _N // _NW
    base = wid * per
    pltpu.sync_copy(rank_hbm.at[pl.ds(base, per)], rank_v)
    pltpu.sync_copy(pred_hbm.at[pl.ds(base, per)], pred_v)
    pltpu.sync_copy(yt_hbm.at[pl.ds(base, per)], yt_v)
    lane = lax.iota(jnp.int32, _SC_LANES)
    idx_refs = (idx0, idx1, idx2, idx3)
    for chunk in range(per // 128):
        for k in range(128 // _SC_LANES):
            off = 128 * chunk + _SC_LANES * k
            r = rank_v[pl.ds(off, _SC_LANES)]
            idx_refs[chunk][pl.ds(_SC_LANES * k, _SC_LANES)] = jnp.where(
                r < _TOPZ, r, _TOPZ)
            val_v[pl.ds(off, _SC_LANES)] = base + off + lane
    copies = []
    for chunk in range(per // 128):
        sl = pl.ds(128 * chunk, 128)
        copies.append(pltpu.async_copy(
            val_v.at[sl], sel_hbm.at[idx_refs[chunk]], sem))
        copies.append(pltpu.async_copy(
            pred_v.at[sl], predg_hbm.at[idx_refs[chunk]], sem))
        copies.append(pltpu.async_copy(
            yt_v.at[sl], ytg_hbm.at[idx_refs[chunk]], sem))
    for c in copies:
        c.wait()


def _sc_scatter(rank, pred, y_true):
    mesh = plsc.VectorSubcoreMesh(core_axis_name="c", subcore_axis_name="s")
    per = _N // _NW
    f = pl.kernel(
        _sc_scatter_body,
        out_type=[jax.ShapeDtypeStruct((_PAD_OUT,), jnp.int32)] * 3,
        mesh=mesh,
        scratch_types=[
            pltpu.VMEM((per,), jnp.int32),
            pltpu.VMEM((per,), jnp.int32),
            pltpu.VMEM((per,), jnp.int32),
            pltpu.VMEM((per,), jnp.int32),
            pltpu.VMEM((128,), jnp.int32),
            pltpu.VMEM((128,), jnp.int32),
            pltpu.VMEM((128,), jnp.int32),
            pltpu.VMEM((128,), jnp.int32),
            pltpu.SemaphoreType.DMA,
        ],
    )
    return f(rank, pred, y_true)


_GC = 8


def _sc_gather_body(x_hbm, sel_hbm, xg_hbm, idx_v, buf0, buf1, gsem, osem):
    wid = lax.axis_index("s") * _NC + lax.axis_index("c")
    rows = _TOPZ // _NW
    base = wid * rows
    pltpu.sync_copy(sel_hbm.at[pl.ds(base, rows)], idx_v)
    bufs = (buf0, buf1)
    nch = rows // _GC
    gcp = [None] * nch
    ocp = [None] * nch
    for c in range(nch):
        b = bufs[c % 2]
        if c >= 2:
            ocp[c - 2].wait()
        gcp[c] = pltpu.async_copy(
            x_hbm.at[idx_v.at[pl.ds(c * _GC, _GC)]], b, gsem)
        if c >= 1:
            gcp[c - 1].wait()
            ocp[c - 1] = pltpu.async_copy(
                bufs[(c - 1) % 2],
                xg_hbm.at[pl.ds(base + (c - 1) * _GC, _GC)], osem)
    gcp[nch - 1].wait()
    ocp[nch - 1] = pltpu.async_copy(
        bufs[(nch - 1) % 2],
        xg_hbm.at[pl.ds(base + (nch - 1) * _GC, _GC)], osem)
    ocp[nch - 2].wait()
    ocp[nch - 1].wait()


def _sc_gather(x, sel_pad):
    mesh = plsc.VectorSubcoreMesh(core_axis_name="c", subcore_axis_name="s")
    f = pl.kernel(
        _sc_gather_body,
        out_type=jax.ShapeDtypeStruct((_TOPZ, _D), jnp.float32),
        mesh=mesh,
        scratch_types=[
            pltpu.VMEM((_TOPZ // _NW,), jnp.int32),
            pltpu.VMEM((_GC, _D), jnp.float32),
            pltpu.VMEM((_GC, _D), jnp.float32),
            pltpu.SemaphoreType.DMA,
            pltpu.SemaphoreType.DMA,
        ],
    )
    return f(x, sel_pad)


_BM = 256


def _loss_body(x_ref, w_ref, pred_ref, yt_ref, loss_ref, acc_ref):
    i = pl.program_id(0)
    logits = jnp.dot(x_ref[...], w_ref[...],
                     preferred_element_type=jnp.float32)
    m = jnp.max(logits, axis=1, keepdims=True)
    e = jnp.exp(logits - m)
    s = jnp.sum(e, axis=1, keepdims=True)
    lse = jnp.log(s) + m
    predg = pred_ref[...]
    cls = lax.broadcasted_iota(jnp.int32, (_BM, _NUM_CLS), 1)
    pick = jnp.sum(jnp.where(cls == predg[:, None], logits, 0.0), axis=1)
    part_loss = jnp.sum(lse[:, 0] - pick)
    part_acc = jnp.sum((predg == yt_ref[...]).astype(jnp.float32))

    @pl.when(i == 0)
    def _():
        loss_ref[...] = jnp.zeros((1, 1), jnp.float32)
        acc_ref[...] = jnp.zeros((1, 1), jnp.float32)

    loss_ref[...] = loss_ref[...] + part_loss
    acc_ref[...] = acc_ref[...] + part_acc

    @pl.when(i == _TOPZ // _BM - 1)
    def _():
        total = loss_ref[...]
        loss_ref[...] = jnp.where(total == 0.0, 0.0,
                                  total / jnp.float32(_TOPZ))
        acc_ref[...] = acc_ref[...] / jnp.float32(_TOPZ)


def _loss_call(xg, W, predg, ytg):
    return pl.pallas_call(
        _loss_body,
        grid=(_TOPZ // _BM,),
        in_specs=[
            pl.BlockSpec((_BM, _D), lambda i: (i, 0)),
            pl.BlockSpec((_D, _NUM_CLS), lambda i: (0, 0)),
            pl.BlockSpec((_BM,), lambda i: (i,)),
            pl.BlockSpec((_BM,), lambda i: (i,)),
        ],
        out_specs=[
            pl.BlockSpec((1, 1), lambda i: (0, 0)),
            pl.BlockSpec((1, 1), lambda i: (0, 0)),
        ],
        out_shape=[
            jax.ShapeDtypeStruct((1, 1), jnp.float32),
            jax.ShapeDtypeStruct((1, 1), jnp.float32),
        ],
    )(xg, W, predg, ytg)


def kernel(x, y_output, y_true, W):
    y_probs = jax.nn.softmax(y_output, axis=1)
    pred = jnp.argmax(y_probs, axis=1).astype(jnp.int32)
    conf = jnp.max(y_probs, axis=1)

    rank2d = _rank_call(conf.reshape(_CR, 128))
    rank = rank2d.reshape(_N)

    sel_pad, predg_pad, ytg_pad = _sc_scatter(rank, pred, y_true)
    selected_idx = sel_pad[:_TOPZ]

    xg = _sc_gather(x, sel_pad)

    loss2, accu2 = _loss_call(xg, W, predg_pad[:_TOPZ], ytg_pad[:_TOPZ])
    return (loss2[0, 0], accu2[0, 0], selected_idx)


# speedup vs baseline: 2.7638x; 2.7540x over previous
import functools

import jax
import jax.numpy as jnp
from jax import lax
from jax.experimental import pallas as pl
from jax.experimental.pallas import tpu as pltpu
from jax.experimental.pallas import tpu_sc as plsc

_NUM_CLS = 1000
_TOPZ = 4096
_N = 16384
_D = 4096

_CR = _N // 128
_RANK_ROWS = 8

_NC = 2
_NS = 16
_NW = _NC * _NS
_SC_LANES = 16

_PAD_OUT = _TOPZ + _N


def _rank_body(ci_ref, cf_ref, rank_ref):
    ib = pl.program_id(0)
    ci3 = ci_ref[...][:, :, None]
    acc0 = jnp.zeros((_RANK_ROWS, 128), jnp.int32)

    def body_before(jr, acc):
        jv3 = cf_ref[pl.ds(jr, 1), :].reshape(1, 1, 128)
        return acc + jnp.sum((jv3 >= ci3).astype(jnp.int32), axis=2)

    def body_after(jr, acc):
        jv3 = cf_ref[pl.ds(jr, 1), :].reshape(1, 1, 128)
        return acc + jnp.sum((jv3 > ci3).astype(jnp.int32), axis=2)

    acc = lax.fori_loop(0, _RANK_ROWS * ib, body_before, acc0)
    acc = lax.fori_loop(_RANK_ROWS * (ib + 1), _CR, body_after, acc)

    r_iota = lax.broadcasted_iota(jnp.int32, (_RANK_ROWS, 128, 1), 0)
    c_iota = lax.broadcasted_iota(jnp.int32, (_RANK_ROWS, 128, 1), 1)
    iidx3 = 128 * (_RANK_ROWS * ib + r_iota) + c_iota
    jlane = lax.broadcasted_iota(jnp.int32, (1, 1, 128), 2)
    for dr in range(_RANK_ROWS):
        jr = _RANK_ROWS * ib + dr
        jv3 = cf_ref[pl.ds(jr, 1), :].reshape(1, 1, 128)
        jidx3 = 128 * jr + jlane
        above = (jv3 > ci3) | ((jv3 == ci3) & (jidx3 < iidx3))
        acc = acc + jnp.sum(above.astype(jnp.int32), axis=2)
    rank_ref[...] = acc


def _rank_call(conf2d):
    return pl.pallas_call(
        _rank_body,
        grid=(_CR // _RANK_ROWS,),
        in_specs=[
            pl.BlockSpec((_RANK_ROWS, 128), lambda i: (i, 0)),
            pl.BlockSpec((_CR, 128), lambda i: (0, 0)),
        ],
        out_specs=pl.BlockSpec((_RANK_ROWS, 128), lambda i: (i, 0)),
        out_shape=jax.ShapeDtypeStruct((_CR, 128), jnp.int32),
    )(conf2d, conf2d)


def _sc_scatter_body(rank_hbm, sel_hbm, rank_v, val_v,
                     idx0, idx1, idx2, idx3, sem):
    wid = lax.axis_index("s") * _NC + lax.axis_index("c")
    per = _N // _NW
    base = wid * per
    pltpu.sync_copy(rank_hbm.at[pl.ds(base, per)], rank_v)
    lane = lax.iota(jnp.int32, _SC_LANES)
    idx_refs = (idx0, idx1, idx2, idx3)
    for chunk in range(per // 128):
        for k in range(128 // _SC_LANES):
            off = 128 * chunk + _SC_LANES * k
            r = rank_v[pl.ds(off, _SC_LANES)]
            own = base + off + lane
            idx_refs[chunk][pl.ds(_SC_LANES * k, _SC_LANES)] = jnp.where(
                r < _TOPZ, r, _TOPZ + own)
            val_v[pl.ds(off, _SC_LANES)] = own
    copies = []
    for chunk in range(per // 128):
        copies.append(pltpu.async_copy(
            val_v.at[pl.ds(128 * chunk, 128)],
            sel_hbm.at[idx_refs[chunk]], sem))
    for c in copies:
        c.wait()


def _sc_scatter(rank):
    mesh = plsc.VectorSubcoreMesh(core_axis_name="c", subcore_axis_name="s")
    per = _N // _NW
    f = pl.kernel(
        _sc_scatter_body,
        out_type=jax.ShapeDtypeStruct((_PAD_OUT,), jnp.int32),
        mesh=mesh,
        scratch_types=[
            pltpu.VMEM((per,), jnp.int32),
            pltpu.VMEM((per,), jnp.int32),
            pltpu.VMEM((128,), jnp.int32),
            pltpu.VMEM((128,), jnp.int32),
            pltpu.VMEM((128,), jnp.int32),
            pltpu.VMEM((128,), jnp.int32),
            pltpu.SemaphoreType.DMA,
        ],
    )
    return f(rank)


_GC = 8


def _sc_gather_body(x_hbm, sel_hbm, pred_hbm, yt_hbm,
                    xg_hbm, predg_hbm, ytg_hbm,
                    idx_v, pg_v, yg_v, buf0, buf1, gsem, osem):
    wid = lax.axis_index("s") * _NC + lax.axis_index("c")
    rows = _TOPZ // _NW
    base = wid * rows
    pltpu.sync_copy(sel_hbm.at[pl.ds(base, rows)], idx_v)
    pgc = pltpu.async_copy(pred_hbm.at[idx_v], pg_v, gsem)
    ygc = pltpu.async_copy(yt_hbm.at[idx_v], yg_v, gsem)
    bufs = (buf0, buf1)
    nch = rows // _GC
    gcp = [None] * nch
    ocp = [None] * nch
    for c in range(nch):
        b = bufs[c % 2]
        if c >= 2:
            ocp[c - 2].wait()
        gcp[c] = pltpu.async_copy(
            x_hbm.at[idx_v.at[pl.ds(c * _GC, _GC)]], b, gsem)
        if c >= 1:
            gcp[c - 1].wait()
            ocp[c - 1] = pltpu.async_copy(
                bufs[(c - 1) % 2],
                xg_hbm.at[pl.ds(base + (c - 1) * _GC, _GC)], osem)
    gcp[nch - 1].wait()
    ocp[nch - 1] = pltpu.async_copy(
        bufs[(nch - 1) % 2],
        xg_hbm.at[pl.ds(base + (nch - 1) * _GC, _GC)], osem)
    ocp[nch - 2].wait()
    ocp[nch - 1].wait()
    pgc.wait()
    ygc.wait()
    pltpu.sync_copy(pg_v, predg_hbm.at[pl.ds(base, rows)])
    pltpu.sync_copy(yg_v, ytg_hbm.at[pl.ds(base, rows)])


def _sc_gather(x, sel, pred, y_true):
    mesh = plsc.VectorSubcoreMesh(core_axis_name="c", subcore_axis_name="s")
    rows = _TOPZ // _NW
    f = pl.kernel(
        _sc_gather_body,
        out_type=[
            jax.ShapeDtypeStruct((_TOPZ, _D), jnp.float32),
            jax.ShapeDtypeStruct((_TOPZ,), jnp.int32),
            jax.ShapeDtypeStruct((_TOPZ,), jnp.int32),
        ],
        mesh=mesh,
        scratch_types=[
            pltpu.VMEM((rows,), jnp.int32),
            pltpu.VMEM((rows,), jnp.int32),
            pltpu.VMEM((rows,), jnp.int32),
            pltpu.VMEM((_GC, _D), jnp.float32),
            pltpu.VMEM((_GC, _D), jnp.float32),
            pltpu.SemaphoreType.DMA,
            pltpu.SemaphoreType.DMA,
        ],
    )
    return f(x, sel, pred, y_true)


_BM = 256


def _loss_body(x_ref, w_ref, pred_ref, yt_ref, loss_ref, acc_ref):
    i = pl.program_id(0)
    logits = jnp.dot(x_ref[...], w_ref[...],
                     preferred_element_type=jnp.float32)
    m = jnp.max(logits, axis=1, keepdims=True)
    e = jnp.exp(logits - m)
    s = jnp.sum(e, axis=1, keepdims=True)
    lse = jnp.log(s) + m
    predg = pred_ref[...]
    cls = lax.broadcasted_iota(jnp.int32, (_BM, _NUM_CLS), 1)
    pick = jnp.sum(jnp.where(cls == predg[:, None], logits, 0.0), axis=1)
    part_loss = jnp.sum(lse[:, 0] - pick)
    part_acc = jnp.sum((predg == yt_ref[...]).astype(jnp.float32))

    @pl.when(i == 0)
    def _():
        loss_ref[...] = jnp.zeros((1, 1), jnp.float32)
        acc_ref[...] = jnp.zeros((1, 1), jnp.float32)

    loss_ref[...] = loss_ref[...] + part_loss
    acc_ref[...] = acc_ref[...] + part_acc

    @pl.when(i == _TOPZ // _BM - 1)
    def _():
        total = loss_ref[...]
        loss_ref[...] = jnp.where(total == 0.0, 0.0,
                                  total / jnp.float32(_TOPZ))
        acc_ref[...] = acc_ref[...] / jnp.float32(_TOPZ)


def _loss_call(xg, W, predg, ytg):
    return pl.pallas_call(
        _loss_body,
        grid=(_TOPZ // _BM,),
        in_specs=[
            pl.BlockSpec((_BM, _D), lambda i: (i, 0)),
            pl.BlockSpec((_D, _NUM_CLS), lambda i: (0, 0)),
            pl.BlockSpec((_BM,), lambda i: (i,)),
            pl.BlockSpec((_BM,), lambda i: (i,)),
        ],
        out_specs=[
            pl.BlockSpec((1, 1), lambda i: (0, 0)),
            pl.BlockSpec((1, 1), lambda i: (0, 0)),
        ],
        out_shape=[
            jax.ShapeDtypeStruct((1, 1), jnp.float32),
            jax.ShapeDtypeStruct((1, 1), jnp.float32),
        ],
    )(xg, W, predg, ytg)


def kernel(x, y_output, y_true, W):
    y_probs = jax.nn.softmax(y_output, axis=1)
    pred = jnp.argmax(y_probs, axis=1).astype(jnp.int32)
    conf = jnp.max(y_probs, axis=1)

    rank2d = _rank_call(conf.reshape(_CR, 128))
    rank = rank2d.reshape(_N)

    sel_pad = _sc_scatter(rank)
    selected_idx = sel_pad[:_TOPZ]

    xg, predg, ytg = _sc_gather(x, sel_pad, pred, y_true)

    loss2, accu2 = _loss_call(xg, W, predg, ytg)
    return (loss2[0, 0], accu2[0, 0], selected_idx)


# speedup vs baseline: 7.2012x; 2.6056x over previous
import functools

import jax
import jax.numpy as jnp
from jax import lax
from jax.experimental import pallas as pl
from jax.experimental.pallas import tpu as pltpu
from jax.experimental.pallas import tpu_sc as plsc

_NUM_CLS = 1000
_TOPZ = 4096
_N = 16384
_D = 4096

_CR = _N // 128
_RANK_ROWS = 8

_NC = 2
_NS = 16
_NW = _NC * _NS
_SC_LANES = 16

_PAD_OUT = _TOPZ + _N


def _rank_body(ci_ref, cf_ref, rank_ref):
    ib = pl.program_id(0)
    ci3 = ci_ref[...][:, :, None]
    acc0 = jnp.zeros((_RANK_ROWS, 128, 128), jnp.float32)

    _G = 8

    def body_before(g, acc3):
        t = acc3
        for d in range(_G):
            jv3 = cf_ref[pl.ds(_G * g + d, 1), :].reshape(1, 1, 128)
            t = t + jnp.where(jv3 >= ci3, 1.0, 0.0)
        return t

    def body_after(g, acc3):
        t = acc3
        for d in range(_G):
            jv3 = cf_ref[pl.ds(_G * g + d, 1), :].reshape(1, 1, 128)
            t = t + jnp.where(jv3 > ci3, 1.0, 0.0)
        return t

    acc3 = lax.fori_loop(0, _RANK_ROWS * ib // _G, body_before, acc0)
    acc3 = lax.fori_loop(_RANK_ROWS * (ib + 1) // _G, _CR // _G, body_after,
                         acc3)

    r_iota = lax.broadcasted_iota(jnp.int32, (_RANK_ROWS, 128, 1), 0)
    c_iota = lax.broadcasted_iota(jnp.int32, (_RANK_ROWS, 128, 1), 1)
    iidx3 = 128 * (_RANK_ROWS * ib + r_iota) + c_iota
    jlane = lax.broadcasted_iota(jnp.int32, (1, 1, 128), 2)
    for dr in range(_RANK_ROWS):
        jr = _RANK_ROWS * ib + dr
        jv3 = cf_ref[pl.ds(jr, 1), :].reshape(1, 1, 128)
        jidx3 = 128 * jr + jlane
        above = (jv3 > ci3) | ((jv3 == ci3) & (jidx3 < iidx3))
        acc3 = acc3 + jnp.where(above, 1.0, 0.0)
    rank_ref[...] = jnp.sum(acc3, axis=2).astype(jnp.int32)


def _rank_call(conf2d):
    return pl.pallas_call(
        _rank_body,
        grid=(_CR // _RANK_ROWS,),
        in_specs=[
            pl.BlockSpec((_RANK_ROWS, 128), lambda i: (i, 0)),
            pl.BlockSpec((_CR, 128), lambda i: (0, 0)),
        ],
        out_specs=pl.BlockSpec((_RANK_ROWS, 128), lambda i: (i, 0)),
        out_shape=jax.ShapeDtypeStruct((_CR, 128), jnp.int32),
    )(conf2d, conf2d)


def _sc_scatter_body(rank_hbm, sel_hbm, rank_v, val_v,
                     idx0, idx1, idx2, idx3, idx4, idx5, idx6, idx7,
                     spbuf, sem):
    c = lax.axis_index("c")
    s = lax.axis_index("s")
    per = _N // _NS
    idx_refs = (idx0, idx1, idx2, idx3, idx4, idx5, idx6, idx7)
    lane = lax.iota(jnp.int32, _SC_LANES)

    @pl.when(c == 0)
    def _():
        base = s * per
        pltpu.sync_copy(rank_hbm.at[pl.ds(base, per)], rank_v)
        for chunk in range(per // 128):
            for k in range(128 // _SC_LANES):
                off = 128 * chunk + _SC_LANES * k
                r = rank_v[pl.ds(off, _SC_LANES)]
                own = base + off + lane
                idx_refs[chunk][pl.ds(_SC_LANES * k, _SC_LANES)] = jnp.where(
                    r < _TOPZ, r, _TOPZ + own)
                val_v[pl.ds(off, _SC_LANES)] = own
        copies = []
        for chunk in range(per // 128):
            copies.append(pltpu.async_copy(
                val_v.at[pl.ds(128 * chunk, 128)],
                spbuf.at[idx_refs[chunk]], sem))
        for cp in copies:
            cp.wait()

    plsc.subcore_barrier()

    @pl.when(c == 0)
    def _():
        out_per = _TOPZ // _NS
        pltpu.sync_copy(spbuf.at[pl.ds(s * out_per, out_per)],
                        sel_hbm.at[pl.ds(s * out_per, out_per)])


def _sc_scatter(rank):
    mesh = plsc.VectorSubcoreMesh(core_axis_name="c", subcore_axis_name="s")
    per = _N // _NS
    f = pl.kernel(
        _sc_scatter_body,
        out_type=jax.ShapeDtypeStruct((_TOPZ,), jnp.int32),
        mesh=mesh,
        scratch_types=[
            pltpu.VMEM((per,), jnp.int32),
            pltpu.VMEM((per,), jnp.int32),
            pltpu.VMEM((128,), jnp.int32),
            pltpu.VMEM((128,), jnp.int32),
            pltpu.VMEM((128,), jnp.int32),
            pltpu.VMEM((128,), jnp.int32),
            pltpu.VMEM((128,), jnp.int32),
            pltpu.VMEM((128,), jnp.int32),
            pltpu.VMEM((128,), jnp.int32),
            pltpu.VMEM((128,), jnp.int32),
            pltpu.VMEM_SHARED((_PAD_OUT,), jnp.int32),
            pltpu.SemaphoreType.DMA,
        ],
    )
    return f(rank)


_GC = 8


_HALF = _TOPZ


def _sc_gather_body(x_hbm, sel_hbm, pred_hbm, yt_hbm,
                    xg_hbm, predg_hbm, ytg_hbm,
                    idx_v, pg_v, yg_v, buf0, buf1, gsem, osem):
    wid = lax.axis_index("s") * _NC + lax.axis_index("c")
    rows = _HALF // _NW
    base = wid * rows
    pltpu.sync_copy(sel_hbm.at[pl.ds(base, rows)], idx_v)
    pgc = pltpu.async_copy(pred_hbm.at[idx_v], pg_v, gsem)
    ygc = pltpu.async_copy(yt_hbm.at[idx_v], yg_v, gsem)
    bufs = (buf0, buf1)
    nch = rows // _GC
    gcp = [None] * nch
    ocp = [None] * nch
    for c in range(nch):
        b = bufs[c % 2]
        if c >= 2:
            ocp[c - 2].wait()
        gcp[c] = pltpu.async_copy(
            x_hbm.at[idx_v.at[pl.ds(c * _GC, _GC)]], b, gsem)
        if c >= 1:
            gcp[c - 1].wait()
            ocp[c - 1] = pltpu.async_copy(
                bufs[(c - 1) % 2],
                xg_hbm.at[pl.ds(base + (c - 1) * _GC, _GC)], osem)
    gcp[nch - 1].wait()
    ocp[nch - 1] = pltpu.async_copy(
        bufs[(nch - 1) % 2],
        xg_hbm.at[pl.ds(base + (nch - 1) * _GC, _GC)], osem)
    ocp[nch - 2].wait()
    ocp[nch - 1].wait()
    pgc.wait()
    ygc.wait()
    pltpu.sync_copy(pg_v, predg_hbm.at[pl.ds(base, rows)])
    pltpu.sync_copy(yg_v, ytg_hbm.at[pl.ds(base, rows)])


def _sc_gather(x, sel, pred, y_true):
    mesh = plsc.VectorSubcoreMesh(core_axis_name="c", subcore_axis_name="s")
    rows = _HALF // _NW
    f = pl.kernel(
        _sc_gather_body,
        out_type=[
            jax.ShapeDtypeStruct((_HALF, _D), jnp.float32),
            jax.ShapeDtypeStruct((_HALF,), jnp.int32),
            jax.ShapeDtypeStruct((_HALF,), jnp.int32),
        ],
        mesh=mesh,
        scratch_types=[
            pltpu.VMEM((rows,), jnp.int32),
            pltpu.VMEM((rows,), jnp.int32),
            pltpu.VMEM((rows,), jnp.int32),
            pltpu.VMEM((_GC, _D), jnp.float32),
            pltpu.VMEM((_GC, _D), jnp.float32),
            pltpu.SemaphoreType.DMA,
            pltpu.SemaphoreType.DMA,
        ],
    )
    return f(x, sel, pred, y_true)


_BM = 256


def _loss_body(x_ref, w_ref, pred_ref, yt_ref, loss_ref, acc_ref):
    i = pl.program_id(0)
    logits = jnp.dot(x_ref[...], w_ref[...],
                     preferred_element_type=jnp.float32)
    m = jnp.max(logits, axis=1, keepdims=True)
    e = jnp.exp(logits - m)
    s = jnp.sum(e, axis=1, keepdims=True)
    lse = jnp.log(s) + m
    predg = pred_ref[...]
    cls = lax.broadcasted_iota(jnp.int32, (_BM, _NUM_CLS), 1)
    pick = jnp.sum(jnp.where(cls == predg[:, None], logits, 0.0), axis=1)
    part_loss = jnp.sum(lse[:, 0] - pick)
    part_acc = jnp.sum((predg == yt_ref[...]).astype(jnp.float32))

    @pl.when(i == 0)
    def _():
        loss_ref[...] = jnp.zeros((1, 1), jnp.float32)
        acc_ref[...] = jnp.zeros((1, 1), jnp.float32)

    loss_ref[...] = loss_ref[...] + part_loss
    acc_ref[...] = acc_ref[...] + part_acc


def _loss_call(xg, W, predg, ytg):
    return pl.pallas_call(
        _loss_body,
        grid=(_HALF // _BM,),
        in_specs=[
            pl.BlockSpec((_BM, _D), lambda i: (i, 0)),
            pl.BlockSpec((_D, _NUM_CLS), lambda i: (0, 0)),
            pl.BlockSpec((_BM,), lambda i: (i,)),
            pl.BlockSpec((_BM,), lambda i: (i,)),
        ],
        out_specs=[
            pl.BlockSpec((1, 1), lambda i: (0, 0)),
            pl.BlockSpec((1, 1), lambda i: (0, 0)),
        ],
        out_shape=[
            jax.ShapeDtypeStruct((1, 1), jnp.float32),
            jax.ShapeDtypeStruct((1, 1), jnp.float32),
        ],
    )(xg, W, predg, ytg)


def kernel(x, y_output, y_true, W):
    y_probs = jax.nn.softmax(y_output, axis=1)
    pred = jnp.argmax(y_probs, axis=1).astype(jnp.int32)
    conf = jnp.max(y_probs, axis=1)

    rank2d = _rank_call(conf.reshape(_CR, 128))
    rank = rank2d.reshape(_N)

    selected_idx = _sc_scatter(rank)

    xg, pg, yt = _sc_gather(x, selected_idx, pred, y_true)
    l1, a1 = _loss_call(xg, W, pg, yt)

    total = l1[0, 0]
    loss = jnp.where(total == 0.0, jnp.float32(0.0),
                     total / jnp.float32(_TOPZ))
    accu = a1[0, 0] / jnp.float32(_TOPZ)
    return (loss, accu, selected_idx)


# speedup vs baseline: 7.2731x; 1.0100x over previous
import functools

import jax
import jax.numpy as jnp
from jax import lax
from jax.experimental import pallas as pl
from jax.experimental.pallas import tpu as pltpu
from jax.experimental.pallas import tpu_sc as plsc

_NUM_CLS = 1000
_TOPZ = 4096
_N = 16384
_D = 4096

_CR = _N // 128
_RANK_ROWS = 8

_NC = 2
_NS = 16
_NW = _NC * _NS
_SC_LANES = 16

_PAD_OUT = _TOPZ + _N


def _rank_body(ci_ref, cf_ref, rank_ref):
    ib = pl.program_id(0)
    ci3 = ci_ref[...][:, :, None]
    acc0 = jnp.zeros((_RANK_ROWS, 128, 128), jnp.float32)

    _G = 8

    def body_before(g, acc3):
        t = acc3
        for d in range(_G):
            jv3 = cf_ref[pl.ds(_G * g + d, 1), :].reshape(1, 1, 128)
            t = t + jnp.where(jv3 >= ci3, 1.0, 0.0)
        return t

    def body_after(g, acc3):
        t = acc3
        for d in range(_G):
            jv3 = cf_ref[pl.ds(_G * g + d, 1), :].reshape(1, 1, 128)
            t = t + jnp.where(jv3 > ci3, 1.0, 0.0)
        return t

    acc3 = lax.fori_loop(0, _RANK_ROWS * ib // _G, body_before, acc0)
    acc3 = lax.fori_loop(_RANK_ROWS * (ib + 1) // _G, _CR // _G, body_after,
                         acc3)

    r_iota = lax.broadcasted_iota(jnp.int32, (_RANK_ROWS, 128, 1), 0)
    c_iota = lax.broadcasted_iota(jnp.int32, (_RANK_ROWS, 128, 1), 1)
    iidx3 = 128 * (_RANK_ROWS * ib + r_iota) + c_iota
    jlane = lax.broadcasted_iota(jnp.int32, (1, 1, 128), 2)
    for dr in range(_RANK_ROWS):
        jr = _RANK_ROWS * ib + dr
        jv3 = cf_ref[pl.ds(jr, 1), :].reshape(1, 1, 128)
        jidx3 = 128 * jr + jlane
        above = (jv3 > ci3) | ((jv3 == ci3) & (jidx3 < iidx3))
        acc3 = acc3 + jnp.where(above, 1.0, 0.0)
    rank_ref[...] = jnp.sum(acc3, axis=2).astype(jnp.int32)


def _rank_call(conf2d):
    return pl.pallas_call(
        _rank_body,
        grid=(_CR // _RANK_ROWS,),
        in_specs=[
            pl.BlockSpec((_RANK_ROWS, 128), lambda i: (i, 0)),
            pl.BlockSpec((_CR, 128), lambda i: (0, 0)),
        ],
        out_specs=pl.BlockSpec((_RANK_ROWS, 128), lambda i: (i, 0)),
        out_shape=jax.ShapeDtypeStruct((_CR, 128), jnp.int32),
    )(conf2d, conf2d)


def _sc_scatter_body(rank_hbm, sel_hbm, rank_v, val_v,
                     idx0, idx1, idx2, idx3, idx4, idx5, idx6, idx7,
                     spbuf, sem):
    c = lax.axis_index("c")
    s = lax.axis_index("s")
    per = _N // _NS
    idx_refs = (idx0, idx1, idx2, idx3, idx4, idx5, idx6, idx7)
    lane = lax.iota(jnp.int32, _SC_LANES)

    @pl.when(c == 0)
    def _():
        base = s * per
        pltpu.sync_copy(rank_hbm.at[pl.ds(base, per)], rank_v)
        for chunk in range(per // 128):
            for k in range(128 // _SC_LANES):
                off = 128 * chunk + _SC_LANES * k
                r = rank_v[pl.ds(off, _SC_LANES)]
                own = base + off + lane
                idx_refs[chunk][pl.ds(_SC_LANES * k, _SC_LANES)] = jnp.where(
                    r < _TOPZ, r, _TOPZ + own)
                val_v[pl.ds(off, _SC_LANES)] = own
        copies = []
        for chunk in range(per // 128):
            copies.append(pltpu.async_copy(
                val_v.at[pl.ds(128 * chunk, 128)],
                spbuf.at[idx_refs[chunk]], sem))
        for cp in copies:
            cp.wait()

    plsc.subcore_barrier()

    @pl.when(c == 0)
    def _():
        out_per = _TOPZ // _NS
        pltpu.sync_copy(spbuf.at[pl.ds(s * out_per, out_per)],
                        sel_hbm.at[pl.ds(s * out_per, out_per)])


def _sc_scatter(rank):
    mesh = plsc.VectorSubcoreMesh(core_axis_name="c", subcore_axis_name="s")
    per = _N // _NS
    f = pl.kernel(
        _sc_scatter_body,
        out_type=jax.ShapeDtypeStruct((_TOPZ,), jnp.int32),
        mesh=mesh,
        scratch_types=[
            pltpu.VMEM((per,), jnp.int32),
            pltpu.VMEM((per,), jnp.int32),
            pltpu.VMEM((128,), jnp.int32),
            pltpu.VMEM((128,), jnp.int32),
            pltpu.VMEM((128,), jnp.int32),
            pltpu.VMEM((128,), jnp.int32),
            pltpu.VMEM((128,), jnp.int32),
            pltpu.VMEM((128,), jnp.int32),
            pltpu.VMEM((128,), jnp.int32),
            pltpu.VMEM((128,), jnp.int32),
            pltpu.VMEM_SHARED((_PAD_OUT,), jnp.int32),
            pltpu.SemaphoreType.DMA,
        ],
    )
    return f(rank)


_GC = 8


_HALF = _TOPZ


def _sc_gather_body(x_hbm, sel_hbm, pred_hbm, yt_hbm,
                    xg_hbm, predg_hbm, ytg_hbm,
                    idx_v, pg_v, yg_v, buf0, buf1, gsem, osem):
    wid = lax.axis_index("s") * _NC + lax.axis_index("c")
    rows = _HALF // _NW
    base = wid * rows
    pltpu.sync_copy(sel_hbm.at[pl.ds(base, rows)], idx_v)
    pgc = pltpu.async_copy(pred_hbm.at[idx_v], pg_v, gsem)
    ygc = pltpu.async_copy(yt_hbm.at[idx_v], yg_v, gsem)
    bufs = (buf0, buf1)
    nch = rows // _GC
    gcp = [None] * nch
    ocp = [None] * nch
    for c in range(nch):
        b = bufs[c % 2]
        if c >= 2:
            ocp[c - 2].wait()
        gcp[c] = pltpu.async_copy(
            x_hbm.at[idx_v.at[pl.ds(c * _GC, _GC)]], b, gsem)
        if c >= 1:
            gcp[c - 1].wait()
            ocp[c - 1] = pltpu.async_copy(
                bufs[(c - 1) % 2],
                xg_hbm.at[pl.ds(base + (c - 1) * _GC, _GC)], osem)
    gcp[nch - 1].wait()
    ocp[nch - 1] = pltpu.async_copy(
        bufs[(nch - 1) % 2],
        xg_hbm.at[pl.ds(base + (nch - 1) * _GC, _GC)], osem)
    ocp[nch - 2].wait()
    ocp[nch - 1].wait()
    pgc.wait()
    ygc.wait()
    pltpu.sync_copy(pg_v, predg_hbm.at[pl.ds(base, rows)])
    pltpu.sync_copy(yg_v, ytg_hbm.at[pl.ds(base, rows)])


def _sc_gather(x, sel, pred, y_true):
    mesh = plsc.VectorSubcoreMesh(core_axis_name="c", subcore_axis_name="s")
    rows = _HALF // _NW
    f = pl.kernel(
        _sc_gather_body,
        out_type=[
            jax.ShapeDtypeStruct((_HALF, _D), jnp.float32),
            jax.ShapeDtypeStruct((_HALF,), jnp.int32),
            jax.ShapeDtypeStruct((_HALF,), jnp.int32),
        ],
        mesh=mesh,
        scratch_types=[
            pltpu.VMEM((rows,), jnp.int32),
            pltpu.VMEM((rows,), jnp.int32),
            pltpu.VMEM((rows,), jnp.int32),
            pltpu.VMEM((_GC, _D), jnp.float32),
            pltpu.VMEM((_GC, _D), jnp.float32),
            pltpu.SemaphoreType.DMA,
            pltpu.SemaphoreType.DMA,
        ],
    )
    return f(x, sel, pred, y_true)


_BM = 256


def _loss_body(x_ref, w_ref, pred_ref, yt_ref, loss_ref, acc_ref):
    i = pl.program_id(0)
    logits = jnp.dot(x_ref[...], w_ref[...],
                     preferred_element_type=jnp.float32)
    m = jnp.max(logits, axis=1, keepdims=True)
    e = jnp.exp(logits - m)
    s = jnp.sum(e, axis=1, keepdims=True)
    lse = jnp.log(s) + m
    predg = pred_ref[...]
    cls = lax.broadcasted_iota(jnp.int32, (_BM, _NUM_CLS), 1)
    pick = jnp.sum(jnp.where(cls == predg[:, None], logits, 0.0), axis=1)
    part_loss = jnp.sum(lse[:, 0] - pick)
    part_acc = jnp.sum((predg == yt_ref[...]).astype(jnp.float32))

    @pl.when(i == 0)
    def _():
        loss_ref[...] = jnp.zeros((1, 1), jnp.float32)
        acc_ref[...] = jnp.zeros((1, 1), jnp.float32)

    loss_ref[...] = loss_ref[...] + part_loss
    acc_ref[...] = acc_ref[...] + part_acc

    @pl.when(i == _HALF // _BM - 1)
    def _():
        total = loss_ref[...]
        loss_ref[...] = jnp.where(total == 0.0, 0.0,
                                  total / jnp.float32(_TOPZ))
        acc_ref[...] = acc_ref[...] / jnp.float32(_TOPZ)


def _loss_call(xg, W, predg, ytg):
    return pl.pallas_call(
        _loss_body,
        grid=(_HALF // _BM,),
        in_specs=[
            pl.BlockSpec((_BM, _D), lambda i: (i, 0)),
            pl.BlockSpec((_D, _NUM_CLS), lambda i: (0, 0)),
            pl.BlockSpec((_BM,), lambda i: (i,)),
            pl.BlockSpec((_BM,), lambda i: (i,)),
        ],
        out_specs=[
            pl.BlockSpec((1, 1), lambda i: (0, 0)),
            pl.BlockSpec((1, 1), lambda i: (0, 0)),
        ],
        out_shape=[
            jax.ShapeDtypeStruct((1, 1), jnp.float32),
            jax.ShapeDtypeStruct((1, 1), jnp.float32),
        ],
    )(xg, W, predg, ytg)


def kernel(x, y_output, y_true, W):
    y_probs = jax.nn.softmax(y_output, axis=1)
    pred = jnp.argmax(y_probs, axis=1).astype(jnp.int32)
    conf = jnp.max(y_probs, axis=1)

    rank2d = _rank_call(conf.reshape(_CR, 128))
    rank = rank2d.reshape(_N)

    selected_idx = _sc_scatter(rank)

    xg, pg, yt = _sc_gather(x, selected_idx, pred, y_true)
    loss2, accu2 = _loss_call(xg, W, pg, yt)
    return (loss2[0, 0], accu2[0, 0], selected_idx)
